# Initial kernel scaffold; baseline (speedup 1.0000x reference)
import jax
import jax.numpy as jnp
from jax.experimental import pallas as pl


def kernel(x, edge_index, edge_weight, W0, b0, W1, b1, W2, b2):
    raise NotImplementedError("write your pallas kernel here")


# speedup vs baseline: 1.2266x; 1.2266x over previous
import functools

import jax
import jax.numpy as jnp
from jax.experimental import pallas as pl

N = 10000
H2 = 128
BM = 200


def _decoder_body(zi_ref, zj_ref, out_ref):
    acc = jax.lax.dot_general(
        zi_ref[...], zj_ref[...],
        (((1,), (1,)), ((), ())),
        preferred_element_type=jnp.float32,
    )
    out_ref[...] = jax.nn.sigmoid(acc)


@jax.jit
def _decoder(z):
    n = z.shape[0]
    grid = (n // BM,)
    return pl.pallas_call(
        _decoder_body,
        grid=grid,
        in_specs=[
            pl.BlockSpec((BM, H2), lambda i: (i, 0)),
            pl.BlockSpec((n, H2), lambda i: (0, 0)),
        ],
        out_specs=pl.BlockSpec((BM, n), lambda i: (i, 0)),
        out_shape=jax.ShapeDtypeStruct((n, n), jnp.float32),
    )(z, z)


def _aggregate(h, src, dst, norm, dinv2):
    msg = h[src] * norm[:, None]
    out = jnp.zeros_like(h).at[dst].add(msg)
    return out + h * dinv2[:, None]


def kernel(x, edge_index, edge_weight, W0, b0, W1, b1, W2, b2):
    src = edge_index[0]
    dst = edge_index[1]
    n = x.shape[0]

    deg = jnp.ones((n,), jnp.float32).at[dst].add(edge_weight)
    dinv = jax.lax.rsqrt(deg)
    norm = dinv[src] * edge_weight * dinv[dst]
    dinv2 = dinv * dinv

    h0 = x @ W0
    h = jax.nn.relu(_aggregate(h0, src, dst, norm, dinv2) + b0)

    m1 = h @ W1
    mean = _aggregate(m1, src, dst, norm, dinv2) + b1
    m2 = h @ W2
    log_std = _aggregate(m2, src, dst, norm, dinv2) + b2

    noise = jax.random.normal(jax.random.key(42), (n, H2), dtype=x.dtype)
    z = mean + noise * jnp.exp(log_std)

    adj_rec = _decoder(z)
    return (adj_rec, z)


# speedup vs baseline: 6.7304x; 5.4868x over previous
import functools

import jax
import jax.numpy as jnp
from jax import lax
from jax.experimental import pallas as pl
from jax.experimental.pallas import tpu as pltpu
from jax.experimental.pallas import tpu_sc as plsc

N = 10000
NPAD = 10240
E = 160000
EPAD = 163840
IN_DIM = 256
H1 = 256
H2 = 128

NC = 2
NS = 16
L = 16

ECHUNK = 128
DEG_NCHUNK = EPAD // (NC * NS * ECHUNK)
AGG_NCHUNK = EPAD // (NS * ECHUNK)
SLAB = NPAD // NS
WB_CHUNK = 128

_sc_mesh = plsc.VectorSubcoreMesh(core_axis_name="c", subcore_axis_name="s")


def _deg_body(dst_hbm, ew_hbm, z1_hbm, deg0_out, deg1_out,
              dst_v, ew_v, zb, shared_deg):
    c = lax.axis_index("c")
    s = lax.axis_index("s")
    wid = c * NS + s

    pltpu.sync_copy(dst_hbm.at[wid], dst_v)
    pltpu.sync_copy(ew_hbm.at[wid], ew_v)

    pltpu.sync_copy(z1_hbm, shared_deg.at[pl.ds(s * SLAB, SLAB)])
    plsc.subcore_barrier()

    def chunk_body(i, _):
        pltpu.sync_copy(ew_v.at[i], shared_deg.at[dst_v.at[i]], add=True)
        return 0
    lax.fori_loop(0, DEG_NCHUNK, chunk_body, 0)
    plsc.subcore_barrier()

    pltpu.sync_copy(shared_deg.at[pl.ds(s * SLAB, SLAB)], zb)

    @pl.when(c == 0)
    def _():
        pltpu.sync_copy(zb, deg0_out.at[pl.ds(s * SLAB, SLAB)])

    @pl.when(c == 1)
    def _():
        pltpu.sync_copy(zb, deg1_out.at[pl.ds(s * SLAB, SLAB)])


@functools.partial(
    pl.kernel,
    out_type=(jax.ShapeDtypeStruct((NPAD,), jnp.float32),
              jax.ShapeDtypeStruct((NPAD,), jnp.float32)),
    mesh=_sc_mesh,
    scratch_types=[
        pltpu.VMEM((DEG_NCHUNK, ECHUNK), jnp.int32),
        pltpu.VMEM((DEG_NCHUNK, ECHUNK), jnp.float32),
        pltpu.VMEM((SLAB,), jnp.float32),
        pltpu.VMEM_SHARED((NPAD,), jnp.float32),
    ],
)
def _deg_call(dst_hbm, ew_hbm, z1_hbm, deg0_out, deg1_out,
              dst_v, ew_v, zb, shared_deg):
    _deg_body(dst_hbm, ew_hbm, z1_hbm, deg0_out, deg1_out,
              dst_v, ew_v, zb, shared_deg)


def _agg_body(ga_hbm, gb_hbm, src_hbm, dst_hbm, ewb_hbm, z2_hbm,
              outa_hbm, outb_hbm, src_c, dst_c, wrow, rows,
              sem, semw, semi, acc):
    c = lax.axis_index("c")
    s = lax.axis_index("s")

    pltpu.sync_copy(z2_hbm, acc.at[pl.ds(s * SLAB, SLAB)])
    plsc.subcore_barrier()

    def scale_rows():
        def row_body(r, _):
            for f in range(H2 // L):
                vals = rows[r, pl.ds(f * L, L)]
                w = wrow[r, pl.ds(f * L, L)]
                rows[r, pl.ds(f * L, L)] = vals * w
            return 0
        lax.fori_loop(0, ECHUNK, row_body, 0)

    def edge_loop(g_ref):
        def body(i, _):
            cw = pltpu.async_copy(ewb_hbm.at[s, i], wrow, semw)
            pltpu.sync_copy(src_hbm.at[s, i], src_c)
            pltpu.async_copy(g_ref.at[src_c], rows, sem).wait()
            pltpu.sync_copy(dst_hbm.at[s, i], dst_c)
            cw.wait()
            scale_rows()
            pltpu.sync_copy(rows, acc.at[dst_c], add=True)
            return 0
        lax.fori_loop(0, AGG_NCHUNK, body, 0)

    @pl.when(c == 0)
    def _():
        edge_loop(ga_hbm)

    @pl.when(c == 1)
    def _():
        edge_loop(gb_hbm)

    plsc.subcore_barrier()

    def wb(out_ref):
        for i in range(SLAB // WB_CHUNK):
            base = s * SLAB + i * WB_CHUNK
            pltpu.sync_copy(acc.at[pl.ds(base, WB_CHUNK)], rows)
            pltpu.sync_copy(rows, out_ref.at[pl.ds(base, WB_CHUNK)])

    @pl.when(c == 0)
    def _():
        wb(outa_hbm)

    @pl.when(c == 1)
    def _():
        wb(outb_hbm)


@functools.partial(
    pl.kernel,
    out_type=(jax.ShapeDtypeStruct((NPAD, H2), jnp.float32),
              jax.ShapeDtypeStruct((NPAD, H2), jnp.float32)),
    mesh=_sc_mesh,
    scratch_types=[
        pltpu.VMEM((ECHUNK,), jnp.int32),
        pltpu.VMEM((ECHUNK,), jnp.int32),
        pltpu.VMEM((ECHUNK, H2), jnp.float32),
        pltpu.VMEM((ECHUNK, H2), jnp.float32),
        pltpu.SemaphoreType.DMA,
        pltpu.SemaphoreType.DMA,
        pltpu.SemaphoreType.DMA,
        pltpu.VMEM_SHARED((NPAD, H2), jnp.float32),
    ],
)
def _agg_call(ga_hbm, gb_hbm, src_hbm, dst_hbm, ewb_hbm, z2_hbm,
              outa_hbm, outb_hbm, src_c, dst_c, wrow, rows,
              sem, semw, semi, acc):
    _agg_body(ga_hbm, gb_hbm, src_hbm, dst_hbm, ewb_hbm, z2_hbm,
              outa_hbm, outb_hbm, src_c, dst_c, wrow, rows,
              sem, semw, semi, acc)


BR = 1000


def _dinv_body(degp_ref, dinv_ref):
    d = degp_ref[0, :] + degp_ref[1, :] + 1.0
    dinv_ref[...] = lax.rsqrt(d)[:, None]


def _dinv_call(deg_p):
    return pl.pallas_call(
        _dinv_body,
        out_shape=jax.ShapeDtypeStruct((NPAD, 1), jnp.float32),
    )(deg_p)


def _t1_body(x_ref, w_ref, dinv_ref, glo_ref, ghi_ref):
    h = jnp.dot(x_ref[...], w_ref[...], preferred_element_type=jnp.float32)
    g = h * dinv_ref[...]
    glo_ref[...] = g[:, :H2]
    ghi_ref[...] = g[:, H2:]


def _t1_call(x, W0, dinv):
    return pl.pallas_call(
        _t1_body,
        grid=(N // BR,),
        in_specs=[
            pl.BlockSpec((BR, IN_DIM), lambda i: (i, 0)),
            pl.BlockSpec((IN_DIM, H1), lambda i: (0, 0)),
            pl.BlockSpec((BR, 1), lambda i: (i, 0)),
        ],
        out_specs=(
            pl.BlockSpec((BR, H2), lambda i: (i, 0)),
            pl.BlockSpec((BR, H2), lambda i: (i, 0)),
        ),
        out_shape=(jax.ShapeDtypeStruct((N, H2), jnp.float32),
                   jax.ShapeDtypeStruct((N, H2), jnp.float32)),
    )(x, W0, dinv)


def _t2_body(alo_ref, ahi_ref, glo_ref, ghi_ref, dinv_ref, b0_ref, wc_ref,
             g1_ref, g2_ref):
    dinv = dinv_ref[...]
    hlo = (alo_ref[...] + glo_ref[...]) * dinv
    hhi = (ahi_ref[...] + ghi_ref[...]) * dinv
    h = jnp.concatenate([hlo, hhi], axis=1) + b0_ref[...]
    h = jnp.maximum(h, 0.0)
    m = jnp.dot(h, wc_ref[...], preferred_element_type=jnp.float32)
    g1_ref[...] = m[:, :H2] * dinv
    g2_ref[...] = m[:, H2:] * dinv


def _t2_call(alo, ahi, glo, ghi, dinv, b0, Wc):
    return pl.pallas_call(
        _t2_body,
        grid=(N // BR,),
        in_specs=[
            pl.BlockSpec((BR, H2), lambda i: (i, 0)),
            pl.BlockSpec((BR, H2), lambda i: (i, 0)),
            pl.BlockSpec((BR, H2), lambda i: (i, 0)),
            pl.BlockSpec((BR, H2), lambda i: (i, 0)),
            pl.BlockSpec((BR, 1), lambda i: (i, 0)),
            pl.BlockSpec((1, H1), lambda i: (0, 0)),
            pl.BlockSpec((H1, 2 * H2), lambda i: (0, 0)),
        ],
        out_specs=(
            pl.BlockSpec((BR, H2), lambda i: (i, 0)),
            pl.BlockSpec((BR, H2), lambda i: (i, 0)),
        ),
        out_shape=(jax.ShapeDtypeStruct((N, H2), jnp.float32),
                   jax.ShapeDtypeStruct((N, H2), jnp.float32)),
    )(alo, ahi, glo, ghi, dinv, b0, Wc)


def _t3_body(a1_ref, g1_ref, a2_ref, g2_ref, dinv_ref, b1_ref, b2_ref,
             noise_ref, z_ref):
    dinv = dinv_ref[...]
    mean = (a1_ref[...] + g1_ref[...]) * dinv + b1_ref[...]
    log_std = (a2_ref[...] + g2_ref[...]) * dinv + b2_ref[...]
    z_ref[...] = mean + noise_ref[...] * jnp.exp(log_std)


def _t3_call(a1, g1, a2, g2, dinv, b1, b2, noise):
    return pl.pallas_call(
        _t3_body,
        grid=(N // BR,),
        in_specs=[
            pl.BlockSpec((BR, H2), lambda i: (i, 0)),
            pl.BlockSpec((BR, H2), lambda i: (i, 0)),
            pl.BlockSpec((BR, H2), lambda i: (i, 0)),
            pl.BlockSpec((BR, H2), lambda i: (i, 0)),
            pl.BlockSpec((BR, 1), lambda i: (i, 0)),
            pl.BlockSpec((1, H2), lambda i: (0, 0)),
            pl.BlockSpec((1, H2), lambda i: (0, 0)),
            pl.BlockSpec((BR, H2), lambda i: (i, 0)),
        ],
        out_specs=pl.BlockSpec((BR, H2), lambda i: (i, 0)),
        out_shape=jax.ShapeDtypeStruct((N, H2), jnp.float32),
    )(a1, g1, a2, g2, dinv, b1, b2, noise)


DEC_BM = 200


def _decoder_body(zi_ref, zj_ref, out_ref):
    acc = jax.lax.dot_general(
        zi_ref[...], zj_ref[...],
        (((1,), (1,)), ((), ())),
        preferred_element_type=jnp.float32,
    )
    out_ref[...] = jax.nn.sigmoid(acc)


def _decoder(z):
    n = z.shape[0]
    return pl.pallas_call(
        _decoder_body,
        grid=(n // DEC_BM,),
        in_specs=[
            pl.BlockSpec((DEC_BM, H2), lambda i: (i, 0)),
            pl.BlockSpec((n, H2), lambda i: (0, 0)),
        ],
        out_specs=pl.BlockSpec((DEC_BM, n), lambda i: (i, 0)),
        out_shape=jax.ShapeDtypeStruct((n, n), jnp.float32),
    )(z, z)


def kernel(x, edge_index, edge_weight, W0, b0, W1, b1, W2, b2):
    src = edge_index[0]
    dst = edge_index[1]
    n = x.shape[0]

    pad = EPAD - E
    pad_idx = (jnp.arange(pad, dtype=jnp.int32) * 37) % N
    src_p = jnp.concatenate([src, pad_idx])
    dst_p = jnp.concatenate([dst, pad_idx])
    ew_p = jnp.concatenate([edge_weight, jnp.zeros((pad,), jnp.float32)])

    dst_deg = dst_p.reshape(NC * NS, DEG_NCHUNK, ECHUNK)
    ew_deg = ew_p.reshape(NC * NS, DEG_NCHUNK, ECHUNK)
    src_agg = src_p.reshape(NS, AGG_NCHUNK, ECHUNK)
    dst_agg = dst_p.reshape(NS, AGG_NCHUNK, ECHUNK)
    ew_bc = jnp.broadcast_to(
        ew_p[:, None], (EPAD, H2)).reshape(NS, AGG_NCHUNK, ECHUNK, H2)

    z1 = jnp.zeros((SLAB,), jnp.float32)
    z2 = jnp.zeros((SLAB, H2), jnp.float32)

    deg0, deg1 = _deg_call(dst_deg, ew_deg, z1)
    dinv_pad = _dinv_call(jnp.stack([deg0, deg1]))
    dinv = dinv_pad[:N]

    g0_lo, g0_hi = _t1_call(x, W0, dinv)
    a0_lo, a0_hi = _agg_call(g0_lo, g0_hi, src_agg, dst_agg, ew_bc, z2)
    a0_lo, a0_hi = a0_lo[:N], a0_hi[:N]

    Wc = jnp.concatenate([W1, W2], axis=1)
    g1, g2 = _t2_call(a0_lo, a0_hi, g0_lo, g0_hi, dinv,
                      b0.reshape(1, H1), Wc)
    a1, a2 = _agg_call(g1, g2, src_agg, dst_agg, ew_bc, z2)
    a1, a2 = a1[:N], a2[:N]

    noise = jax.random.normal(jax.random.key(42), (n, H2), dtype=x.dtype)
    z = _t3_call(a1, g1, a2, g2, dinv,
                 b1.reshape(1, H2), b2.reshape(1, H2), noise)

    adj_rec = _decoder(z)
    return (adj_rec, z)
